# Initial kernel scaffold; baseline (speedup 1.0000x reference)
import jax
import jax.numpy as jnp
from jax.experimental import pallas as pl


def kernel(x, edge_index, edge_type, batch, W1, root1, b1, W2, root2, b2, lin1_w, lin1_b, lin2_w, lin2_b):
    raise NotImplementedError("write your pallas kernel here")


# speedup vs baseline: 9.8651x; 9.8651x over previous
import functools

import jax
import jax.numpy as jnp
from jax import lax
from jax.experimental import pallas as pl
from jax.experimental.pallas import tpu as pltpu
from jax.experimental.pallas import tpu_sc as plsc

N = 10000
E = 320000
H = 128
R = 8
C = 16
G = 16

NC, NS, L = 2, 16, 16
NW = NC * NS
N_PAD = 10240
ACC_ROWS = N_PAD + 16
RPT = N_PAD // NS
RN = R * N_PAD
CNT_ROWS = RN + 128
CPT = RN // NS
EPW = 10240
E_PAD = NW * EPW
EC = 2048
GB = 128
NB = EPW // GB

BM = 256
NBLK = N_PAD // BM

_MESH = plsc.VectorSubcoreMesh(core_axis_name="c", subcore_axis_name="s",
                               num_cores=NC, num_subcores=NS)


def _sc_cnt_body(dsts, typs, zc, cnt_out,
                 d_v, t_v, widx, ones_v, acc):
    cid = lax.axis_index("c")
    sid = lax.axis_index("s")
    wid = sid * NC + cid
    ebase = wid * EPW

    @pl.when(sid == NS - 1)
    def _():
        pltpu.sync_copy(zc, acc.at[pl.ds(sid * CPT, CPT + 128)])

    @pl.when(sid != NS - 1)
    def _():
        pltpu.sync_copy(zc.at[pl.ds(0, CPT)], acc.at[pl.ds(sid * CPT, CPT)])

    for j in range(GB // L):
        ones_v[pl.ds(j * L, L)] = jnp.ones((L,), jnp.float32)
    plsc.subcore_barrier()

    def bbody(b, carry):
        eb = ebase + b * GB
        pltpu.sync_copy(dsts.at[pl.ds(eb, GB)], d_v)
        pltpu.sync_copy(typs.at[pl.ds(eb, GB)], t_v)
        for j in range(GB // L):
            sl = pl.ds(j * L, L)
            widx[0, sl] = t_v[sl] * N_PAD + d_v[sl]
        pltpu.sync_copy(ones_v, acc.at[widx.at[0]], add=True)
        return carry

    lax.fori_loop(0, NB, bbody, 0)
    plsc.subcore_barrier()

    pltpu.sync_copy(acc.at[pl.ds(sid * CPT, CPT)],
                    cnt_out.at[cid, pl.ds(sid * CPT, CPT)])


_sc_cnt = pl.kernel(
    _sc_cnt_body,
    out_type=jax.ShapeDtypeStruct((NC, RN), jnp.float32),
    mesh=_MESH,
    scratch_types=[
        pltpu.VMEM((GB,), jnp.int32),
        pltpu.VMEM((GB,), jnp.int32),
        pltpu.VMEM((1, GB), jnp.int32),
        pltpu.VMEM((GB,), jnp.float32),
        pltpu.VMEM_SHARED((CNT_ROWS,), jnp.float32),
    ],
)


def _sc_agg_body(tbl, rc, srcs, dsts, typs, zs, agg_out,
                 s_v, d_v, t_v, gidx, widx, didx, w_v, rows_v,
                 acc, gsem, ssem):
    cid = lax.axis_index("c")
    sid = lax.axis_index("s")
    wid = sid * NC + cid
    ebase = wid * EPW

    @pl.when(sid == NS - 1)
    def _():
        pltpu.sync_copy(zs, acc.at[pl.ds(sid * RPT, RPT + 16)])

    @pl.when(sid != NS - 1)
    def _():
        pltpu.sync_copy(zs.at[pl.ds(0, RPT)], acc.at[pl.ds(sid * RPT, RPT)])

    plsc.subcore_barrier()

    ones = jnp.ones((L,), jnp.float32)

    def bbody(b, carry):
        eb = ebase + b * GB
        pltpu.sync_copy(srcs.at[pl.ds(eb, GB)], s_v)
        pltpu.sync_copy(dsts.at[pl.ds(eb, GB)], d_v)
        pltpu.sync_copy(typs.at[pl.ds(eb, GB)], t_v)
        for j in range(GB // L):
            sl = pl.ds(j * L, L)
            t = jnp.minimum(t_v[sl], R - 1)
            gidx[0, sl] = t * N_PAD + s_v[sl]
            widx[0, sl] = jnp.minimum(t_v[sl] * N_PAD + d_v[sl], RN - 1)
            didx[0, sl] = d_v[sl]
        pltpu.async_copy(tbl.at[gidx.at[0]], rows_v, gsem).wait()
        pltpu.async_copy(rc.at[widx.at[0]], w_v, ssem).wait()

        def gbody(g, carry):
            wvec = w_v[pl.ds(g * L, L)]
            for jj in range(L):
                j = g * L + jj
                wv = wvec[jj] * ones
                for ccp in range(H // L):
                    sl = pl.ds(ccp * L, L)
                    rows_v[j, sl] = rows_v[j, sl] * wv
            return carry

        lax.fori_loop(0, GB // L, gbody, 0)
        pltpu.sync_copy(rows_v, acc.at[didx.at[0]], add=True)
        return carry

    lax.fori_loop(0, NB, bbody, 0)
    plsc.subcore_barrier()

    pltpu.sync_copy(acc.at[pl.ds(sid * RPT, RPT)],
                    agg_out.at[cid, pl.ds(sid * RPT, RPT)])


_sc_agg = pl.kernel(
    _sc_agg_body,
    out_type=jax.ShapeDtypeStruct((NC, N_PAD, H), jnp.float32),
    mesh=_MESH,
    scratch_types=[
        pltpu.VMEM((GB,), jnp.int32),
        pltpu.VMEM((GB,), jnp.int32),
        pltpu.VMEM((GB,), jnp.int32),
        pltpu.VMEM((1, GB), jnp.int32),
        pltpu.VMEM((1, GB), jnp.int32),
        pltpu.VMEM((1, GB), jnp.int32),
        pltpu.VMEM((GB,), jnp.float32),
        pltpu.VMEM((GB, H), jnp.float32),
        pltpu.VMEM_SHARED((ACC_ROWS, H), jnp.float32),
        pltpu.SemaphoreType.DMA,
        pltpu.SemaphoreType.DMA,
    ],
)


def _tc_recip(cnt):
    rpb = 64

    def body(c_ref, o_ref):
        c = c_ref[0] + c_ref[1]
        o_ref[...] = 1.0 / jnp.maximum(c, 1.0)

    return pl.pallas_call(
        body,
        grid=(RN // 128 // rpb,),
        in_specs=[pl.BlockSpec((NC, rpb, 128), lambda i: (0, i, 0))],
        out_specs=pl.BlockSpec((rpb, 128), lambda i: (i, 0)),
        out_shape=jax.ShapeDtypeStruct((RN // 128, 128), jnp.float32),
    )(cnt)


def _tc_ht(h_pad, W):
    def body(h_ref, w_ref, o_ref):
        for r in range(R):
            o_ref[r] = jnp.dot(h_ref[...], w_ref[r],
                               preferred_element_type=jnp.float32)

    return pl.pallas_call(
        body,
        grid=(NBLK,),
        in_specs=[
            pl.BlockSpec((BM, H), lambda i: (i, 0)),
            pl.BlockSpec((R, H, H), lambda i: (0, 0, 0)),
        ],
        out_specs=pl.BlockSpec((R, BM, H), lambda i: (0, i, 0)),
        out_shape=jax.ShapeDtypeStruct((R, N_PAD, H), jnp.float32),
    )(h_pad, W)


def _tc_layer(h_pad, agg, root, b):
    def body(h_ref, a_ref, root_ref, b_ref, o_ref):
        acc = jnp.dot(h_ref[...], root_ref[...],
                      preferred_element_type=jnp.float32) + b_ref[...]
        acc = acc + a_ref[0] + a_ref[1]
        o_ref[...] = jnp.maximum(acc, 0.0)

    return pl.pallas_call(
        body,
        grid=(NBLK,),
        in_specs=[
            pl.BlockSpec((BM, H), lambda i: (i, 0)),
            pl.BlockSpec((NC, BM, H), lambda i: (0, i, 0)),
            pl.BlockSpec((H, H), lambda i: (0, 0)),
            pl.BlockSpec((1, H), lambda i: (0, 0)),
        ],
        out_specs=pl.BlockSpec((BM, H), lambda i: (i, 0)),
        out_shape=jax.ShapeDtypeStruct((N_PAD, H), jnp.float32),
    )(h_pad, agg, root, b.reshape(1, H))


def _tc_final(h1, agg, root2, b2, batch3, lin1_w, lin1_b, lin2_w, lin2_b):
    def body(h_ref, a_ref, root_ref, b_ref, bt_ref,
             l1w_ref, l1b_ref, l2w_ref, l2b_ref, o_ref, pool, pcnt):
        i = pl.program_id(0)
        acc = jnp.dot(h_ref[...], root_ref[...],
                      preferred_element_type=jnp.float32) + b_ref[...]
        h2 = jnp.maximum(acc + a_ref[0] + a_ref[1], 0.0)

        bt = bt_ref[0, 0, :]
        onehot = (bt[:, None] ==
                  lax.broadcasted_iota(jnp.int32, (BM, G), 1)
                  ).astype(jnp.float32)

        @pl.when(i == 0)
        def _():
            pool[...] = jnp.zeros((G, H), jnp.float32)
            pcnt[...] = jnp.zeros((G, H), jnp.float32)

        dn = (((0,), (0,)), ((), ()))
        pool[...] += lax.dot_general(onehot, h2, dn,
                                     preferred_element_type=jnp.float32)
        pcnt[...] += lax.dot_general(onehot, jnp.ones((BM, H), jnp.float32),
                                     dn, preferred_element_type=jnp.float32)

        @pl.when(i == NBLK - 1)
        def _():
            pooled = pool[...] / jnp.maximum(pcnt[...], 1.0)
            hh = jnp.maximum(
                jnp.dot(pooled, l1w_ref[...],
                        preferred_element_type=jnp.float32) + l1b_ref[...],
                0.0)
            o_ref[...] = jnp.dot(hh, l2w_ref[...],
                                 preferred_element_type=jnp.float32) + l2b_ref[...]

    return pl.pallas_call(
        body,
        grid=(NBLK,),
        in_specs=[
            pl.BlockSpec((BM, H), lambda i: (i, 0)),
            pl.BlockSpec((NC, BM, H), lambda i: (0, i, 0)),
            pl.BlockSpec((H, H), lambda i: (0, 0)),
            pl.BlockSpec((1, H), lambda i: (0, 0)),
            pl.BlockSpec((1, 1, BM), lambda i: (i, 0, 0)),
            pl.BlockSpec((H, H), lambda i: (0, 0)),
            pl.BlockSpec((1, H), lambda i: (0, 0)),
            pl.BlockSpec((H, C), lambda i: (0, 0)),
            pl.BlockSpec((1, C), lambda i: (0, 0)),
        ],
        out_specs=pl.BlockSpec((G, C), lambda i: (0, 0)),
        out_shape=jax.ShapeDtypeStruct((G, C), jnp.float32),
        scratch_shapes=[
            pltpu.VMEM((G, H), jnp.float32),
            pltpu.VMEM((G, H), jnp.float32),
        ],
    )(h1, agg, root2, b2.reshape(1, H), batch3,
      lin1_w, lin1_b.reshape(1, H), lin2_w, lin2_b.reshape(1, C))


def kernel(x, edge_index, edge_type, batch, W1, root1, b1, W2, root2, b2,
           lin1_w, lin1_b, lin2_w, lin2_b):
    x_pad = jnp.pad(x, ((0, N_PAD - N), (0, 0)))
    epad = E_PAD - E
    srcs = jnp.pad(edge_index[0], (0, epad))
    dsts = jnp.pad(edge_index[1], (0, epad))
    typs = jnp.pad(edge_type, (0, epad), constant_values=R)
    dsts_agg = jnp.pad(edge_index[1], (0, epad), constant_values=N_PAD)
    batch3 = jnp.pad(batch, (0, N_PAD - N),
                     constant_values=G).reshape(NBLK, 1, BM)
    zs = jnp.zeros((RPT + 16, H), jnp.float32)
    zc = jnp.zeros((CPT + 128,), jnp.float32)

    cnt = _sc_cnt(dsts, typs, zc)
    rc = _tc_recip(cnt.reshape(NC, RN // 128, 128)).reshape(RN)

    ht1 = _tc_ht(x_pad, W1).reshape(RN, H)
    agg1 = _sc_agg(ht1, rc, srcs, dsts_agg, typs, zs)
    h1 = _tc_layer(x_pad, agg1, root1, b1)

    ht2 = _tc_ht(h1, W2).reshape(RN, H)
    agg2 = _sc_agg(ht2, rc, srcs, dsts_agg, typs, zs)
    return _tc_final(h1, agg2, root2, b2, batch3,
                     lin1_w, lin1_b, lin2_w, lin2_b)


# speedup vs baseline: 12.5922x; 1.2764x over previous
import functools

import jax
import jax.numpy as jnp
from jax import lax
from jax.experimental import pallas as pl
from jax.experimental.pallas import tpu as pltpu
from jax.experimental.pallas import tpu_sc as plsc

N = 10000
E = 320000
H = 128
R = 8
C = 16
G = 16

NC, NS, L = 2, 16, 16
NW = NC * NS
N_PAD = 10240
ACC_ROWS = N_PAD + 16
RPT = N_PAD // NS
RN = R * N_PAD
CNT_ROWS = RN + 128
CPT = RN // NS
EPW = 10240
E_PAD = NW * EPW
EC = 2048
GB = 128
NB = EPW // GB

BM = 256
NBLK = N_PAD // BM

_MESH = plsc.VectorSubcoreMesh(core_axis_name="c", subcore_axis_name="s",
                               num_cores=NC, num_subcores=NS)


def _sc_cnt_body(dsts, typs, zc, cnt_out,
                 d_v, t_v, widx, ones_v, acc, ssem):
    cid = lax.axis_index("c")
    sid = lax.axis_index("s")
    wid = sid * NC + cid
    ebase = wid * EPW

    @pl.when(sid == NS - 1)
    def _():
        pltpu.sync_copy(zc, acc.at[pl.ds(sid * CPT, CPT + 128)])

    @pl.when(sid != NS - 1)
    def _():
        pltpu.sync_copy(zc.at[pl.ds(0, CPT)], acc.at[pl.ds(sid * CPT, CPT)])

    for j in range(GB // L):
        ones_v[pl.ds(j * L, L)] = jnp.ones((L,), jnp.float32)
    plsc.subcore_barrier()

    def bbody(g, carry):
        descs = []
        for s in range(2):
            eb = ebase + (g * 2 + s) * GB
            pltpu.sync_copy(dsts.at[pl.ds(eb, GB)], d_v)
            pltpu.sync_copy(typs.at[pl.ds(eb, GB)], t_v)
            for j in range(GB // L):
                sl = pl.ds(j * L, L)
                widx[s, sl] = t_v[sl] * N_PAD + d_v[sl]
            descs.append(pltpu.async_copy(ones_v, acc.at[widx.at[s]],
                                          ssem, add=True))
        for d in descs:
            d.wait()
        return carry

    lax.fori_loop(0, NB // 2, bbody, 0)
    plsc.subcore_barrier()

    pltpu.sync_copy(acc.at[pl.ds(sid * CPT, CPT)],
                    cnt_out.at[cid, pl.ds(sid * CPT, CPT)])


_sc_cnt = pl.kernel(
    _sc_cnt_body,
    out_type=jax.ShapeDtypeStruct((NC, RN), jnp.float32),
    mesh=_MESH,
    scratch_types=[
        pltpu.VMEM((GB,), jnp.int32),
        pltpu.VMEM((GB,), jnp.int32),
        pltpu.VMEM((2, GB), jnp.int32),
        pltpu.VMEM((GB,), jnp.float32),
        pltpu.VMEM_SHARED((CNT_ROWS,), jnp.float32),
        pltpu.SemaphoreType.DMA,
    ],
)


def _sc_agg_body(tbl, rc, srcs, dsts, typs, zs, agg_out,
                 s_v, d_v, t_v, gidx, widx, didx, w_v, rows_v,
                 acc, gsem, ssem, wsem):
    cid = lax.axis_index("c")
    sid = lax.axis_index("s")
    wid = sid * NC + cid
    ebase = wid * EPW

    @pl.when(sid == NS - 1)
    def _():
        pltpu.sync_copy(zs, acc.at[pl.ds(sid * RPT, RPT + 16)])

    @pl.when(sid != NS - 1)
    def _():
        pltpu.sync_copy(zs.at[pl.ds(0, RPT)], acc.at[pl.ds(sid * RPT, RPT)])

    plsc.subcore_barrier()

    ones = jnp.ones((L,), jnp.float32)

    def bbody(gp, carry):
        gdesc = []
        for s in range(2):
            eb = ebase + (gp * 2 + s) * GB
            pltpu.sync_copy(srcs.at[pl.ds(eb, GB)], s_v)
            pltpu.sync_copy(dsts.at[pl.ds(eb, GB)], d_v)
            pltpu.sync_copy(typs.at[pl.ds(eb, GB)], t_v)
            for j in range(GB // L):
                sl = pl.ds(j * L, L)
                t = jnp.minimum(t_v[sl], R - 1)
                gidx[s, sl] = t * N_PAD + s_v[sl]
                widx[s, sl] = jnp.minimum(t_v[sl] * N_PAD + d_v[sl], RN - 1)
                didx[s, sl] = d_v[sl]
            rslot = rows_v.at[pl.ds(s * GB, GB)]
            gdesc.append((
                pltpu.async_copy(tbl.at[gidx.at[s]], rslot, gsem),
                pltpu.async_copy(rc.at[widx.at[s]],
                                 w_v.at[pl.ds(s * GB, GB)], ssem),
            ))

        sdesc = []
        for s in range(2):
            dg, dw = gdesc[s]
            dg.wait()
            dw.wait()

            def gbody(g, carry, s=s):
                wvec = w_v[pl.ds(s * GB + g * L, L)]
                for jj in range(L):
                    j = s * GB + g * L + jj
                    wv = wvec[jj] * ones
                    for ccp in range(H // L):
                        sl = pl.ds(ccp * L, L)
                        rows_v[j, sl] = rows_v[j, sl] * wv
                return carry

            lax.fori_loop(0, GB // L, gbody, 0)
            sdesc.append(pltpu.async_copy(rows_v.at[pl.ds(s * GB, GB)],
                                          acc.at[didx.at[s]], wsem, add=True))
        for d in sdesc:
            d.wait()
        return carry

    lax.fori_loop(0, NB // 2, bbody, 0)
    plsc.subcore_barrier()

    pltpu.sync_copy(acc.at[pl.ds(sid * RPT, RPT)],
                    agg_out.at[cid, pl.ds(sid * RPT, RPT)])


_sc_agg = pl.kernel(
    _sc_agg_body,
    out_type=jax.ShapeDtypeStruct((NC, N_PAD, H), jnp.float32),
    mesh=_MESH,
    scratch_types=[
        pltpu.VMEM((GB,), jnp.int32),
        pltpu.VMEM((GB,), jnp.int32),
        pltpu.VMEM((GB,), jnp.int32),
        pltpu.VMEM((2, GB), jnp.int32),
        pltpu.VMEM((2, GB), jnp.int32),
        pltpu.VMEM((2, GB), jnp.int32),
        pltpu.VMEM((2 * GB,), jnp.float32),
        pltpu.VMEM((2 * GB, H), jnp.float32),
        pltpu.VMEM_SHARED((ACC_ROWS, H), jnp.float32),
        pltpu.SemaphoreType.DMA,
        pltpu.SemaphoreType.DMA,
        pltpu.SemaphoreType.DMA,
    ],
)


def _tc_recip(cnt):
    rpb = 64

    def body(c_ref, o_ref):
        c = c_ref[0] + c_ref[1]
        o_ref[...] = 1.0 / jnp.maximum(c, 1.0)

    return pl.pallas_call(
        body,
        grid=(RN // 128 // rpb,),
        in_specs=[pl.BlockSpec((NC, rpb, 128), lambda i: (0, i, 0))],
        out_specs=pl.BlockSpec((rpb, 128), lambda i: (i, 0)),
        out_shape=jax.ShapeDtypeStruct((RN // 128, 128), jnp.float32),
    )(cnt)


def _tc_ht(h_pad, W):
    def body(h_ref, w_ref, o_ref):
        for r in range(R):
            o_ref[r] = jnp.dot(h_ref[...], w_ref[r],
                               preferred_element_type=jnp.float32)

    return pl.pallas_call(
        body,
        grid=(NBLK,),
        in_specs=[
            pl.BlockSpec((BM, H), lambda i: (i, 0)),
            pl.BlockSpec((R, H, H), lambda i: (0, 0, 0)),
        ],
        out_specs=pl.BlockSpec((R, BM, H), lambda i: (0, i, 0)),
        out_shape=jax.ShapeDtypeStruct((R, N_PAD, H), jnp.float32),
    )(h_pad, W)


def _tc_layer(h_pad, agg, root, b):
    def body(h_ref, a_ref, root_ref, b_ref, o_ref):
        acc = jnp.dot(h_ref[...], root_ref[...],
                      preferred_element_type=jnp.float32) + b_ref[...]
        acc = acc + a_ref[0] + a_ref[1]
        o_ref[...] = jnp.maximum(acc, 0.0)

    return pl.pallas_call(
        body,
        grid=(NBLK,),
        in_specs=[
            pl.BlockSpec((BM, H), lambda i: (i, 0)),
            pl.BlockSpec((NC, BM, H), lambda i: (0, i, 0)),
            pl.BlockSpec((H, H), lambda i: (0, 0)),
            pl.BlockSpec((1, H), lambda i: (0, 0)),
        ],
        out_specs=pl.BlockSpec((BM, H), lambda i: (i, 0)),
        out_shape=jax.ShapeDtypeStruct((N_PAD, H), jnp.float32),
    )(h_pad, agg, root, b.reshape(1, H))


def _tc_final(h1, agg, root2, b2, batch3, lin1_w, lin1_b, lin2_w, lin2_b):
    def body(h_ref, a_ref, root_ref, b_ref, bt_ref,
             l1w_ref, l1b_ref, l2w_ref, l2b_ref, o_ref, pool, pcnt):
        i = pl.program_id(0)
        acc = jnp.dot(h_ref[...], root_ref[...],
                      preferred_element_type=jnp.float32) + b_ref[...]
        h2 = jnp.maximum(acc + a_ref[0] + a_ref[1], 0.0)

        bt = bt_ref[0, 0, :]
        onehot = (bt[:, None] ==
                  lax.broadcasted_iota(jnp.int32, (BM, G), 1)
                  ).astype(jnp.float32)

        @pl.when(i == 0)
        def _():
            pool[...] = jnp.zeros((G, H), jnp.float32)
            pcnt[...] = jnp.zeros((G, H), jnp.float32)

        dn = (((0,), (0,)), ((), ()))
        pool[...] += lax.dot_general(onehot, h2, dn,
                                     preferred_element_type=jnp.float32)
        pcnt[...] += lax.dot_general(onehot, jnp.ones((BM, H), jnp.float32),
                                     dn, preferred_element_type=jnp.float32)

        @pl.when(i == NBLK - 1)
        def _():
            pooled = pool[...] / jnp.maximum(pcnt[...], 1.0)
            hh = jnp.maximum(
                jnp.dot(pooled, l1w_ref[...],
                        preferred_element_type=jnp.float32) + l1b_ref[...],
                0.0)
            o_ref[...] = jnp.dot(hh, l2w_ref[...],
                                 preferred_element_type=jnp.float32) + l2b_ref[...]

    return pl.pallas_call(
        body,
        grid=(NBLK,),
        in_specs=[
            pl.BlockSpec((BM, H), lambda i: (i, 0)),
            pl.BlockSpec((NC, BM, H), lambda i: (0, i, 0)),
            pl.BlockSpec((H, H), lambda i: (0, 0)),
            pl.BlockSpec((1, H), lambda i: (0, 0)),
            pl.BlockSpec((1, 1, BM), lambda i: (i, 0, 0)),
            pl.BlockSpec((H, H), lambda i: (0, 0)),
            pl.BlockSpec((1, H), lambda i: (0, 0)),
            pl.BlockSpec((H, C), lambda i: (0, 0)),
            pl.BlockSpec((1, C), lambda i: (0, 0)),
        ],
        out_specs=pl.BlockSpec((G, C), lambda i: (0, 0)),
        out_shape=jax.ShapeDtypeStruct((G, C), jnp.float32),
        scratch_shapes=[
            pltpu.VMEM((G, H), jnp.float32),
            pltpu.VMEM((G, H), jnp.float32),
        ],
    )(h1, agg, root2, b2.reshape(1, H), batch3,
      lin1_w, lin1_b.reshape(1, H), lin2_w, lin2_b.reshape(1, C))


def kernel(x, edge_index, edge_type, batch, W1, root1, b1, W2, root2, b2,
           lin1_w, lin1_b, lin2_w, lin2_b):
    x_pad = jnp.pad(x, ((0, N_PAD - N), (0, 0)))
    epad = E_PAD - E
    srcs = jnp.pad(edge_index[0], (0, epad))
    dsts = jnp.pad(edge_index[1], (0, epad))
    typs = jnp.pad(edge_type, (0, epad), constant_values=R)
    dsts_agg = jnp.pad(edge_index[1], (0, epad), constant_values=N_PAD)
    batch3 = jnp.pad(batch, (0, N_PAD - N),
                     constant_values=G).reshape(NBLK, 1, BM)
    zs = jnp.zeros((RPT + 16, H), jnp.float32)
    zc = jnp.zeros((CPT + 128,), jnp.float32)

    cnt = _sc_cnt(dsts, typs, zc)
    rc = _tc_recip(cnt.reshape(NC, RN // 128, 128)).reshape(RN)

    ht1 = _tc_ht(x_pad, W1).reshape(RN, H)
    agg1 = _sc_agg(ht1, rc, srcs, dsts_agg, typs, zs)
    h1 = _tc_layer(x_pad, agg1, root1, b1)

    ht2 = _tc_ht(h1, W2).reshape(RN, H)
    agg2 = _sc_agg(ht2, rc, srcs, dsts_agg, typs, zs)
    return _tc_final(h1, agg2, root2, b2, batch3,
                     lin1_w, lin1_b, lin2_w, lin2_b)


# speedup vs baseline: 12.5954x; 1.0003x over previous
import functools

import jax
import jax.numpy as jnp
from jax import lax
from jax.experimental import pallas as pl
from jax.experimental.pallas import tpu as pltpu
from jax.experimental.pallas import tpu_sc as plsc

N = 10000
E = 320000
H = 128
R = 8
C = 16
G = 16

NC, NS, L = 2, 16, 16
NW = NC * NS
N_PAD = 10240
ACC_ROWS = N_PAD + 16
RPT = N_PAD // NS
RN = R * N_PAD
CNT_ROWS = RN + 128
CPT = RN // NS
EPW = 10240
E_PAD = NW * EPW
EC = 2048
GB = 128
NB = EPW // GB

BM = 256
NBLK = N_PAD // BM

_MESH = plsc.VectorSubcoreMesh(core_axis_name="c", subcore_axis_name="s",
                               num_cores=NC, num_subcores=NS)


def _sc_cnt_body(dsts, typs, zc, cnt_out,
                 d_v, t_v, widx, ones_v, acc, ssem):
    cid = lax.axis_index("c")
    sid = lax.axis_index("s")
    wid = sid * NC + cid
    ebase = wid * EPW

    @pl.when(sid == NS - 1)
    def _():
        pltpu.sync_copy(zc, acc.at[pl.ds(sid * CPT, CPT + 128)])

    @pl.when(sid != NS - 1)
    def _():
        pltpu.sync_copy(zc.at[pl.ds(0, CPT)], acc.at[pl.ds(sid * CPT, CPT)])

    for j in range(GB // L):
        ones_v[pl.ds(j * L, L)] = jnp.ones((L,), jnp.float32)
    plsc.subcore_barrier()

    def bbody(g, carry):
        @pl.when(g > 0)
        def _():
            for s in range(2):
                pltpu.make_async_copy(ones_v, acc.at[widx.at[s]],
                                      ssem).wait()

        for s in range(2):
            eb = ebase + (g * 2 + s) * GB
            pltpu.sync_copy(dsts.at[pl.ds(eb, GB)], d_v)
            pltpu.sync_copy(typs.at[pl.ds(eb, GB)], t_v)
            for j in range(GB // L):
                sl = pl.ds(j * L, L)
                widx[s, sl] = t_v[sl] * N_PAD + d_v[sl]
            pltpu.async_copy(ones_v, acc.at[widx.at[s]], ssem, add=True)
        return carry

    lax.fori_loop(0, NB // 2, bbody, 0)
    for s in range(2):
        pltpu.make_async_copy(ones_v, acc.at[widx.at[s]], ssem).wait()
    plsc.subcore_barrier()

    pltpu.sync_copy(acc.at[pl.ds(sid * CPT, CPT)],
                    cnt_out.at[cid, pl.ds(sid * CPT, CPT)])


_sc_cnt = pl.kernel(
    _sc_cnt_body,
    out_type=jax.ShapeDtypeStruct((NC, RN), jnp.float32),
    mesh=_MESH,
    scratch_types=[
        pltpu.VMEM((GB,), jnp.int32),
        pltpu.VMEM((GB,), jnp.int32),
        pltpu.VMEM((2, GB), jnp.int32),
        pltpu.VMEM((GB,), jnp.float32),
        pltpu.VMEM_SHARED((CNT_ROWS,), jnp.float32),
        pltpu.SemaphoreType.DMA,
    ],
)


def _sc_agg_body(tbl, rc, srcs, dsts, typs, zs, agg_out,
                 s_v, d_v, t_v, gidx, widx, didx, w_v, rows_v,
                 acc, gsem, ssem, wsem):
    cid = lax.axis_index("c")
    sid = lax.axis_index("s")
    wid = sid * NC + cid
    ebase = wid * EPW

    @pl.when(sid == NS - 1)
    def _():
        pltpu.sync_copy(zs, acc.at[pl.ds(sid * RPT, RPT + 16)])

    @pl.when(sid != NS - 1)
    def _():
        pltpu.sync_copy(zs.at[pl.ds(0, RPT)], acc.at[pl.ds(sid * RPT, RPT)])

    plsc.subcore_barrier()

    ones = jnp.ones((L,), jnp.float32)

    def bbody(gp, carry):
        @pl.when(gp > 0)
        def _():
            for s in range(2):
                pltpu.make_async_copy(rows_v.at[pl.ds(s * GB, GB)],
                                      acc.at[didx.at[s]], wsem).wait()

        gdesc = []
        for s in range(2):
            eb = ebase + (gp * 2 + s) * GB
            pltpu.sync_copy(srcs.at[pl.ds(eb, GB)], s_v)
            pltpu.sync_copy(dsts.at[pl.ds(eb, GB)], d_v)
            pltpu.sync_copy(typs.at[pl.ds(eb, GB)], t_v)
            for j in range(GB // L):
                sl = pl.ds(j * L, L)
                t = jnp.minimum(t_v[sl], R - 1)
                gidx[s, sl] = t * N_PAD + s_v[sl]
                widx[s, sl] = jnp.minimum(t_v[sl] * N_PAD + d_v[sl], RN - 1)
                didx[s, sl] = d_v[sl]
            rslot = rows_v.at[pl.ds(s * GB, GB)]
            gdesc.append((
                pltpu.async_copy(tbl.at[gidx.at[s]], rslot, gsem),
                pltpu.async_copy(rc.at[widx.at[s]],
                                 w_v.at[pl.ds(s * GB, GB)], ssem),
            ))

        for s in range(2):
            dg, dw = gdesc[s]
            dg.wait()
            dw.wait()

            def gbody(g, carry, s=s):
                wvec = w_v[pl.ds(s * GB + g * L, L)]
                for jj in range(L):
                    j = s * GB + g * L + jj
                    wv = wvec[jj] * ones
                    for ccp in range(H // L):
                        sl = pl.ds(ccp * L, L)
                        rows_v[j, sl] = rows_v[j, sl] * wv
                return carry

            lax.fori_loop(0, GB // L, gbody, 0)
            pltpu.async_copy(rows_v.at[pl.ds(s * GB, GB)],
                             acc.at[didx.at[s]], wsem, add=True)
        return carry

    lax.fori_loop(0, NB // 2, bbody, 0)

    for s in range(2):
        pltpu.make_async_copy(rows_v.at[pl.ds(s * GB, GB)],
                              acc.at[didx.at[s]], wsem).wait()
    plsc.subcore_barrier()

    pltpu.sync_copy(acc.at[pl.ds(sid * RPT, RPT)],
                    agg_out.at[cid, pl.ds(sid * RPT, RPT)])


_sc_agg = pl.kernel(
    _sc_agg_body,
    out_type=jax.ShapeDtypeStruct((NC, N_PAD, H), jnp.float32),
    mesh=_MESH,
    scratch_types=[
        pltpu.VMEM((GB,), jnp.int32),
        pltpu.VMEM((GB,), jnp.int32),
        pltpu.VMEM((GB,), jnp.int32),
        pltpu.VMEM((2, GB), jnp.int32),
        pltpu.VMEM((2, GB), jnp.int32),
        pltpu.VMEM((2, GB), jnp.int32),
        pltpu.VMEM((2 * GB,), jnp.float32),
        pltpu.VMEM((2 * GB, H), jnp.float32),
        pltpu.VMEM_SHARED((ACC_ROWS, H), jnp.float32),
        pltpu.SemaphoreType.DMA,
        pltpu.SemaphoreType.DMA,
        pltpu.SemaphoreType.DMA,
    ],
)


def _tc_recip(cnt):
    rpb = 64

    def body(c_ref, o_ref):
        c = c_ref[0] + c_ref[1]
        o_ref[...] = 1.0 / jnp.maximum(c, 1.0)

    return pl.pallas_call(
        body,
        grid=(RN // 128 // rpb,),
        in_specs=[pl.BlockSpec((NC, rpb, 128), lambda i: (0, i, 0))],
        out_specs=pl.BlockSpec((rpb, 128), lambda i: (i, 0)),
        out_shape=jax.ShapeDtypeStruct((RN // 128, 128), jnp.float32),
    )(cnt)


def _tc_ht(h_pad, W):
    def body(h_ref, w_ref, o_ref):
        for r in range(R):
            o_ref[r] = jnp.dot(h_ref[...], w_ref[r],
                               preferred_element_type=jnp.float32)

    return pl.pallas_call(
        body,
        grid=(NBLK,),
        in_specs=[
            pl.BlockSpec((BM, H), lambda i: (i, 0)),
            pl.BlockSpec((R, H, H), lambda i: (0, 0, 0)),
        ],
        out_specs=pl.BlockSpec((R, BM, H), lambda i: (0, i, 0)),
        out_shape=jax.ShapeDtypeStruct((R, N_PAD, H), jnp.float32),
    )(h_pad, W)


def _tc_layer(h_pad, agg, root, b):
    def body(h_ref, a_ref, root_ref, b_ref, o_ref):
        acc = jnp.dot(h_ref[...], root_ref[...],
                      preferred_element_type=jnp.float32) + b_ref[...]
        acc = acc + a_ref[0] + a_ref[1]
        o_ref[...] = jnp.maximum(acc, 0.0)

    return pl.pallas_call(
        body,
        grid=(NBLK,),
        in_specs=[
            pl.BlockSpec((BM, H), lambda i: (i, 0)),
            pl.BlockSpec((NC, BM, H), lambda i: (0, i, 0)),
            pl.BlockSpec((H, H), lambda i: (0, 0)),
            pl.BlockSpec((1, H), lambda i: (0, 0)),
        ],
        out_specs=pl.BlockSpec((BM, H), lambda i: (i, 0)),
        out_shape=jax.ShapeDtypeStruct((N_PAD, H), jnp.float32),
    )(h_pad, agg, root, b.reshape(1, H))


def _tc_final(h1, agg, root2, b2, batch3, lin1_w, lin1_b, lin2_w, lin2_b):
    def body(h_ref, a_ref, root_ref, b_ref, bt_ref,
             l1w_ref, l1b_ref, l2w_ref, l2b_ref, o_ref, pool, pcnt):
        i = pl.program_id(0)
        acc = jnp.dot(h_ref[...], root_ref[...],
                      preferred_element_type=jnp.float32) + b_ref[...]
        h2 = jnp.maximum(acc + a_ref[0] + a_ref[1], 0.0)

        bt = bt_ref[0, 0, :]
        onehot = (bt[:, None] ==
                  lax.broadcasted_iota(jnp.int32, (BM, G), 1)
                  ).astype(jnp.float32)

        @pl.when(i == 0)
        def _():
            pool[...] = jnp.zeros((G, H), jnp.float32)
            pcnt[...] = jnp.zeros((G, H), jnp.float32)

        dn = (((0,), (0,)), ((), ()))
        pool[...] += lax.dot_general(onehot, h2, dn,
                                     preferred_element_type=jnp.float32)
        pcnt[...] += lax.dot_general(onehot, jnp.ones((BM, H), jnp.float32),
                                     dn, preferred_element_type=jnp.float32)

        @pl.when(i == NBLK - 1)
        def _():
            pooled = pool[...] / jnp.maximum(pcnt[...], 1.0)
            hh = jnp.maximum(
                jnp.dot(pooled, l1w_ref[...],
                        preferred_element_type=jnp.float32) + l1b_ref[...],
                0.0)
            o_ref[...] = jnp.dot(hh, l2w_ref[...],
                                 preferred_element_type=jnp.float32) + l2b_ref[...]

    return pl.pallas_call(
        body,
        grid=(NBLK,),
        in_specs=[
            pl.BlockSpec((BM, H), lambda i: (i, 0)),
            pl.BlockSpec((NC, BM, H), lambda i: (0, i, 0)),
            pl.BlockSpec((H, H), lambda i: (0, 0)),
            pl.BlockSpec((1, H), lambda i: (0, 0)),
            pl.BlockSpec((1, 1, BM), lambda i: (i, 0, 0)),
            pl.BlockSpec((H, H), lambda i: (0, 0)),
            pl.BlockSpec((1, H), lambda i: (0, 0)),
            pl.BlockSpec((H, C), lambda i: (0, 0)),
            pl.BlockSpec((1, C), lambda i: (0, 0)),
        ],
        out_specs=pl.BlockSpec((G, C), lambda i: (0, 0)),
        out_shape=jax.ShapeDtypeStruct((G, C), jnp.float32),
        scratch_shapes=[
            pltpu.VMEM((G, H), jnp.float32),
            pltpu.VMEM((G, H), jnp.float32),
        ],
    )(h1, agg, root2, b2.reshape(1, H), batch3,
      lin1_w, lin1_b.reshape(1, H), lin2_w, lin2_b.reshape(1, C))


def kernel(x, edge_index, edge_type, batch, W1, root1, b1, W2, root2, b2,
           lin1_w, lin1_b, lin2_w, lin2_b):
    x_pad = jnp.pad(x, ((0, N_PAD - N), (0, 0)))
    epad = E_PAD - E
    srcs = jnp.pad(edge_index[0], (0, epad))
    dsts = jnp.pad(edge_index[1], (0, epad))
    typs = jnp.pad(edge_type, (0, epad), constant_values=R)
    dsts_agg = jnp.pad(edge_index[1], (0, epad), constant_values=N_PAD)
    batch3 = jnp.pad(batch, (0, N_PAD - N),
                     constant_values=G).reshape(NBLK, 1, BM)
    zs = jnp.zeros((RPT + 16, H), jnp.float32)
    zc = jnp.zeros((CPT + 128,), jnp.float32)

    cnt = _sc_cnt(dsts, typs, zc)
    rc = _tc_recip(cnt.reshape(NC, RN // 128, 128)).reshape(RN)

    ht1 = _tc_ht(x_pad, W1).reshape(RN, H)
    agg1 = _sc_agg(ht1, rc, srcs, dsts_agg, typs, zs)
    h1 = _tc_layer(x_pad, agg1, root1, b1)

    ht2 = _tc_ht(h1, W2).reshape(RN, H)
    agg2 = _sc_agg(ht2, rc, srcs, dsts_agg, typs, zs)
    return _tc_final(h1, agg2, root2, b2, batch3,
                     lin1_w, lin1_b, lin2_w, lin2_b)


# speedup vs baseline: 14.3748x; 1.1413x over previous
import functools

import jax
import jax.numpy as jnp
from jax import lax
from jax.experimental import pallas as pl
from jax.experimental.pallas import tpu as pltpu
from jax.experimental.pallas import tpu_sc as plsc

N = 10000
E = 320000
H = 128
R = 8
C = 16
G = 16

NC, NS, L = 2, 16, 16
NW = NC * NS
N_PAD = 10240
ACC_ROWS = N_PAD + 16
RPT = N_PAD // NS
RN = R * N_PAD
CNT_ROWS = RN + 128
CPT = RN // NS
EPW = 10240
E_PAD = NW * EPW
GB = 128
NB = EPW // GB
NB0, NB1 = 110, 50
EPW0, EPW1 = NB0 * GB, NB1 * GB

BM = 256
NBLK = N_PAD // BM

_MESH = plsc.VectorSubcoreMesh(core_axis_name="c", subcore_axis_name="s",
                               num_cores=NC, num_subcores=NS)


def _sc_cnt_body(dsts, typs, zc, cnt_out,
                 d_v, t_v, widx, ones_v, acc, ssem):
    cid = lax.axis_index("c")
    sid = lax.axis_index("s")
    wid = sid * NC + cid
    ebase = wid * EPW

    @pl.when(sid == NS - 1)
    def _():
        pltpu.sync_copy(zc, acc.at[pl.ds(sid * CPT, CPT + 128)])

    @pl.when(sid != NS - 1)
    def _():
        pltpu.sync_copy(zc.at[pl.ds(0, CPT)], acc.at[pl.ds(sid * CPT, CPT)])

    for j in range(GB // L):
        ones_v[pl.ds(j * L, L)] = jnp.ones((L,), jnp.float32)
    plsc.subcore_barrier()

    def bbody(g, carry):
        @pl.when(g > 0)
        def _():
            for s in range(2):
                pltpu.make_async_copy(ones_v, acc.at[widx.at[s]],
                                      ssem).wait()

        for s in range(2):
            eb = ebase + (g * 2 + s) * GB
            pltpu.sync_copy(dsts.at[pl.ds(eb, GB)], d_v)
            pltpu.sync_copy(typs.at[pl.ds(eb, GB)], t_v)
            for j in range(GB // L):
                sl = pl.ds(j * L, L)
                widx[s, sl] = t_v[sl] * N_PAD + d_v[sl]
            pltpu.async_copy(ones_v, acc.at[widx.at[s]], ssem, add=True)
        return carry

    lax.fori_loop(0, NB // 2, bbody, 0)
    for s in range(2):
        pltpu.make_async_copy(ones_v, acc.at[widx.at[s]], ssem).wait()
    plsc.subcore_barrier()

    pltpu.sync_copy(acc.at[pl.ds(sid * CPT, CPT)],
                    cnt_out.at[cid, pl.ds(sid * CPT, CPT)])


_sc_cnt = pl.kernel(
    _sc_cnt_body,
    out_type=jax.ShapeDtypeStruct((NC, RN), jnp.float32),
    mesh=_MESH,
    scratch_types=[
        pltpu.VMEM((GB,), jnp.int32),
        pltpu.VMEM((GB,), jnp.int32),
        pltpu.VMEM((2, GB), jnp.int32),
        pltpu.VMEM((GB,), jnp.float32),
        pltpu.VMEM_SHARED((CNT_ROWS,), jnp.float32),
        pltpu.SemaphoreType.DMA,
    ],
)


def _sc_agg_body(tbl, rc, srcs, dsts, typs, zs, agg_out,
                 s_v, d_v, t_v, gidx, widx, didx, w_v, rows_v,
                 acc, gsem, ssem, wsem):
    cid = lax.axis_index("c")
    sid = lax.axis_index("s")
    ebase = jnp.where(cid == 0, sid * EPW0, NS * EPW0 + sid * EPW1)
    npairs = jnp.where(cid == 0, NB0 // 2, NB1 // 2)

    @pl.when(sid == NS - 1)
    def _():
        pltpu.sync_copy(zs, acc.at[pl.ds(sid * RPT, RPT + 16)])

    @pl.when(sid != NS - 1)
    def _():
        pltpu.sync_copy(zs.at[pl.ds(0, RPT)], acc.at[pl.ds(sid * RPT, RPT)])

    plsc.subcore_barrier()

    ones = jnp.ones((L,), jnp.float32)

    def bbody(gp, carry):
        @pl.when(gp > 0)
        def _():
            for s in range(2):
                pltpu.make_async_copy(rows_v.at[pl.ds(s * GB, GB)],
                                      acc.at[didx.at[s]], wsem).wait()

        gdesc = []
        for s in range(2):
            eb = ebase + (gp * 2 + s) * GB
            pltpu.sync_copy(srcs.at[pl.ds(eb, GB)], s_v)
            pltpu.sync_copy(dsts.at[pl.ds(eb, GB)], d_v)
            pltpu.sync_copy(typs.at[pl.ds(eb, GB)], t_v)
            for j in range(GB // L):
                sl = pl.ds(j * L, L)
                t = jnp.minimum(t_v[sl], R - 1)
                gidx[s, sl] = t * N_PAD + s_v[sl]
                widx[s, sl] = jnp.minimum(t_v[sl] * N_PAD + d_v[sl], RN - 1)
                didx[s, sl] = d_v[sl]
            rslot = rows_v.at[pl.ds(s * GB, GB)]
            gdesc.append((
                pltpu.async_copy(tbl.at[gidx.at[s]], rslot, gsem),
                pltpu.async_copy(rc.at[widx.at[s]],
                                 w_v.at[pl.ds(s * GB, GB)], ssem),
            ))

        for s in range(2):
            dg, dw = gdesc[s]
            dg.wait()
            dw.wait()

            def gbody(g, carry, s=s):
                wvec = w_v[pl.ds(s * GB + g * L, L)]
                for jj in range(L):
                    j = s * GB + g * L + jj
                    wv = wvec[jj] * ones
                    for ccp in range(H // L):
                        sl = pl.ds(ccp * L, L)
                        rows_v[j, sl] = rows_v[j, sl] * wv
                return carry

            lax.fori_loop(0, GB // L, gbody, 0)
            pltpu.async_copy(rows_v.at[pl.ds(s * GB, GB)],
                             acc.at[didx.at[s]], wsem, add=True)
        return carry

    lax.fori_loop(0, npairs, bbody, 0)

    for s in range(2):
        pltpu.make_async_copy(rows_v.at[pl.ds(s * GB, GB)],
                              acc.at[didx.at[s]], wsem).wait()
    plsc.subcore_barrier()

    pltpu.sync_copy(acc.at[pl.ds(sid * RPT, RPT)],
                    agg_out.at[cid, pl.ds(sid * RPT, RPT)])


_sc_agg = pl.kernel(
    _sc_agg_body,
    out_type=jax.ShapeDtypeStruct((NC, N_PAD, H), jnp.float32),
    mesh=_MESH,
    scratch_types=[
        pltpu.VMEM((GB,), jnp.int32),
        pltpu.VMEM((GB,), jnp.int32),
        pltpu.VMEM((GB,), jnp.int32),
        pltpu.VMEM((2, GB), jnp.int32),
        pltpu.VMEM((2, GB), jnp.int32),
        pltpu.VMEM((2, GB), jnp.int32),
        pltpu.VMEM((2 * GB,), jnp.float32),
        pltpu.VMEM((2 * GB, H), jnp.float32),
        pltpu.VMEM_SHARED((ACC_ROWS, H), jnp.float32),
        pltpu.SemaphoreType.DMA,
        pltpu.SemaphoreType.DMA,
        pltpu.SemaphoreType.DMA,
    ],
)


def _tc_recip(cnt):
    rpb = 64

    def body(c_ref, o_ref):
        c = c_ref[0] + c_ref[1]
        o_ref[...] = 1.0 / jnp.maximum(c, 1.0)

    return pl.pallas_call(
        body,
        grid=(RN // 128 // rpb,),
        in_specs=[pl.BlockSpec((NC, rpb, 128), lambda i: (0, i, 0))],
        out_specs=pl.BlockSpec((rpb, 128), lambda i: (i, 0)),
        out_shape=jax.ShapeDtypeStruct((RN // 128, 128), jnp.float32),
    )(cnt)


def _tc_ht(h_pad, W):
    def body(h_ref, w_ref, o_ref):
        for r in range(R):
            o_ref[r] = jnp.dot(h_ref[...], w_ref[r],
                               preferred_element_type=jnp.float32)

    return pl.pallas_call(
        body,
        grid=(NBLK,),
        in_specs=[
            pl.BlockSpec((BM, H), lambda i: (i, 0)),
            pl.BlockSpec((R, H, H), lambda i: (0, 0, 0)),
        ],
        out_specs=pl.BlockSpec((R, BM, H), lambda i: (0, i, 0)),
        out_shape=jax.ShapeDtypeStruct((R, N_PAD, H), jnp.float32),
    )(h_pad, W)


def _tc_layer(h_pad, agg, root, b):
    def body(h_ref, a_ref, root_ref, b_ref, o_ref):
        acc = jnp.dot(h_ref[...], root_ref[...],
                      preferred_element_type=jnp.float32) + b_ref[...]
        acc = acc + a_ref[0] + a_ref[1]
        o_ref[...] = jnp.maximum(acc, 0.0)

    return pl.pallas_call(
        body,
        grid=(NBLK,),
        in_specs=[
            pl.BlockSpec((BM, H), lambda i: (i, 0)),
            pl.BlockSpec((NC, BM, H), lambda i: (0, i, 0)),
            pl.BlockSpec((H, H), lambda i: (0, 0)),
            pl.BlockSpec((1, H), lambda i: (0, 0)),
        ],
        out_specs=pl.BlockSpec((BM, H), lambda i: (i, 0)),
        out_shape=jax.ShapeDtypeStruct((N_PAD, H), jnp.float32),
    )(h_pad, agg, root, b.reshape(1, H))


def _tc_final(h1, agg, root2, b2, batch3, lin1_w, lin1_b, lin2_w, lin2_b):
    def body(h_ref, a_ref, root_ref, b_ref, bt_ref,
             l1w_ref, l1b_ref, l2w_ref, l2b_ref, o_ref, pool, pcnt):
        i = pl.program_id(0)
        acc = jnp.dot(h_ref[...], root_ref[...],
                      preferred_element_type=jnp.float32) + b_ref[...]
        h2 = jnp.maximum(acc + a_ref[0] + a_ref[1], 0.0)

        bt = bt_ref[0, 0, :]
        onehot = (bt[:, None] ==
                  lax.broadcasted_iota(jnp.int32, (BM, G), 1)
                  ).astype(jnp.float32)

        @pl.when(i == 0)
        def _():
            pool[...] = jnp.zeros((G, H), jnp.float32)
            pcnt[...] = jnp.zeros((G, H), jnp.float32)

        dn = (((0,), (0,)), ((), ()))
        pool[...] += lax.dot_general(onehot, h2, dn,
                                     preferred_element_type=jnp.float32)
        pcnt[...] += lax.dot_general(onehot, jnp.ones((BM, H), jnp.float32),
                                     dn, preferred_element_type=jnp.float32)

        @pl.when(i == NBLK - 1)
        def _():
            pooled = pool[...] / jnp.maximum(pcnt[...], 1.0)
            hh = jnp.maximum(
                jnp.dot(pooled, l1w_ref[...],
                        preferred_element_type=jnp.float32) + l1b_ref[...],
                0.0)
            o_ref[...] = jnp.dot(hh, l2w_ref[...],
                                 preferred_element_type=jnp.float32) + l2b_ref[...]

    return pl.pallas_call(
        body,
        grid=(NBLK,),
        in_specs=[
            pl.BlockSpec((BM, H), lambda i: (i, 0)),
            pl.BlockSpec((NC, BM, H), lambda i: (0, i, 0)),
            pl.BlockSpec((H, H), lambda i: (0, 0)),
            pl.BlockSpec((1, H), lambda i: (0, 0)),
            pl.BlockSpec((1, 1, BM), lambda i: (i, 0, 0)),
            pl.BlockSpec((H, H), lambda i: (0, 0)),
            pl.BlockSpec((1, H), lambda i: (0, 0)),
            pl.BlockSpec((H, C), lambda i: (0, 0)),
            pl.BlockSpec((1, C), lambda i: (0, 0)),
        ],
        out_specs=pl.BlockSpec((G, C), lambda i: (0, 0)),
        out_shape=jax.ShapeDtypeStruct((G, C), jnp.float32),
        scratch_shapes=[
            pltpu.VMEM((G, H), jnp.float32),
            pltpu.VMEM((G, H), jnp.float32),
        ],
    )(h1, agg, root2, b2.reshape(1, H), batch3,
      lin1_w, lin1_b.reshape(1, H), lin2_w, lin2_b.reshape(1, C))


def kernel(x, edge_index, edge_type, batch, W1, root1, b1, W2, root2, b2,
           lin1_w, lin1_b, lin2_w, lin2_b):
    x_pad = jnp.pad(x, ((0, N_PAD - N), (0, 0)))
    epad = E_PAD - E
    srcs = jnp.pad(edge_index[0], (0, epad))
    dsts = jnp.pad(edge_index[1], (0, epad))
    typs = jnp.pad(edge_type, (0, epad), constant_values=R)
    dsts_agg = jnp.pad(edge_index[1], (0, epad), constant_values=N_PAD)
    batch3 = jnp.pad(batch, (0, N_PAD - N),
                     constant_values=G).reshape(NBLK, 1, BM)
    zs = jnp.zeros((RPT + 16, H), jnp.float32)
    zc = jnp.zeros((CPT + 128,), jnp.float32)

    cnt = _sc_cnt(dsts, typs, zc)
    rc = _tc_recip(cnt.reshape(NC, RN // 128, 128)).reshape(RN)

    ht1 = _tc_ht(x_pad, W1).reshape(RN, H)
    agg1 = _sc_agg(ht1, rc, srcs, dsts_agg, typs, zs)
    h1 = _tc_layer(x_pad, agg1, root1, b1)

    ht2 = _tc_ht(h1, W2).reshape(RN, H)
    agg2 = _sc_agg(ht2, rc, srcs, dsts_agg, typs, zs)
    return _tc_final(h1, agg2, root2, b2, batch3,
                     lin1_w, lin1_b, lin2_w, lin2_b)


# speedup vs baseline: 15.2947x; 1.0640x over previous
import functools

import jax
import jax.numpy as jnp
from jax import lax
from jax.experimental import pallas as pl
from jax.experimental.pallas import tpu as pltpu
from jax.experimental.pallas import tpu_sc as plsc

N = 10000
E = 320000
H = 128
R = 8
C = 16
G = 16

NC, NS, L = 2, 16, 16
NW = NC * NS
N_PAD = 10240
ACC_ROWS = N_PAD + 16
RPT = N_PAD // NS
RN = R * N_PAD
CNT_ROWS = RN + 128
CPT = RN // NS
EPW = 10240
E_PAD = NW * EPW
GB = 128
NB = EPW // GB
NB0, NB1 = 120, 40
EPW0, EPW1 = NB0 * GB, NB1 * GB

BM = 256
NBLK = N_PAD // BM

_MESH = plsc.VectorSubcoreMesh(core_axis_name="c", subcore_axis_name="s",
                               num_cores=NC, num_subcores=NS)


def _sc_cnt_body(dsts, typs, zc, cnt_out,
                 d_v, t_v, widx, ones_v, acc, ssem):
    cid = lax.axis_index("c")
    sid = lax.axis_index("s")
    wid = sid * NC + cid
    ebase = wid * EPW

    @pl.when(sid == NS - 1)
    def _():
        pltpu.sync_copy(zc, acc.at[pl.ds(sid * CPT, CPT + 128)])

    @pl.when(sid != NS - 1)
    def _():
        pltpu.sync_copy(zc.at[pl.ds(0, CPT)], acc.at[pl.ds(sid * CPT, CPT)])

    for j in range(GB // L):
        ones_v[pl.ds(j * L, L)] = jnp.ones((L,), jnp.float32)
    plsc.subcore_barrier()

    def bbody(g, carry):
        @pl.when(g > 0)
        def _():
            for s in range(2):
                pltpu.make_async_copy(ones_v, acc.at[widx.at[s]],
                                      ssem).wait()

        for s in range(2):
            eb = ebase + (g * 2 + s) * GB
            pltpu.sync_copy(dsts.at[pl.ds(eb, GB)], d_v)
            pltpu.sync_copy(typs.at[pl.ds(eb, GB)], t_v)
            for j in range(GB // L):
                sl = pl.ds(j * L, L)
                widx[s, sl] = t_v[sl] * N_PAD + d_v[sl]
            pltpu.async_copy(ones_v, acc.at[widx.at[s]], ssem, add=True)
        return carry

    lax.fori_loop(0, NB // 2, bbody, 0)
    for s in range(2):
        pltpu.make_async_copy(ones_v, acc.at[widx.at[s]], ssem).wait()
    plsc.subcore_barrier()

    pltpu.sync_copy(acc.at[pl.ds(sid * CPT, CPT)],
                    cnt_out.at[cid, pl.ds(sid * CPT, CPT)])


_sc_cnt = pl.kernel(
    _sc_cnt_body,
    out_type=jax.ShapeDtypeStruct((NC, RN), jnp.float32),
    mesh=_MESH,
    scratch_types=[
        pltpu.VMEM((GB,), jnp.int32),
        pltpu.VMEM((GB,), jnp.int32),
        pltpu.VMEM((2, GB), jnp.int32),
        pltpu.VMEM((GB,), jnp.float32),
        pltpu.VMEM_SHARED((CNT_ROWS,), jnp.float32),
        pltpu.SemaphoreType.DMA,
    ],
)


def _sc_agg_body(tbl, rc, srcs, dsts, typs, zs, agg_out,
                 s_v, d_v, t_v, gidx, widx, didx, w_v, rows_v,
                 acc, gsem, ssem, wsem, esem):
    cid = lax.axis_index("c")
    sid = lax.axis_index("s")
    ebase = jnp.where(cid == 0, sid * EPW0, NS * EPW0 + sid * EPW1)
    npairs = jnp.where(cid == 0, NB0 // 2, NB1 // 2)

    @pl.when(sid == NS - 1)
    def _():
        pltpu.sync_copy(zs, acc.at[pl.ds(sid * RPT, RPT + 16)])

    @pl.when(sid != NS - 1)
    def _():
        pltpu.sync_copy(zs.at[pl.ds(0, RPT)], acc.at[pl.ds(sid * RPT, RPT)])

    plsc.subcore_barrier()

    ones = jnp.ones((L,), jnp.float32)

    pltpu.async_copy(srcs.at[pl.ds(ebase, 2 * GB)], s_v.at[0], esem)
    pltpu.async_copy(dsts.at[pl.ds(ebase, 2 * GB)], d_v.at[0], esem)
    pltpu.async_copy(typs.at[pl.ds(ebase, 2 * GB)], t_v.at[0], esem)

    def bbody(gp, carry):
        pslot = gp & 1

        @pl.when(gp > 0)
        def _():
            for s in range(2):
                pltpu.make_async_copy(rows_v.at[pl.ds(s * GB, GB)],
                                      acc.at[didx.at[s]], wsem).wait()

        pltpu.make_async_copy(srcs.at[pl.ds(ebase, 2 * GB)],
                              s_v.at[pslot], esem).wait()
        pltpu.make_async_copy(dsts.at[pl.ds(ebase, 2 * GB)],
                              d_v.at[pslot], esem).wait()
        pltpu.make_async_copy(typs.at[pl.ds(ebase, 2 * GB)],
                              t_v.at[pslot], esem).wait()

        @pl.when(gp + 1 < npairs)
        def _():
            nb = ebase + (gp + 1) * 2 * GB
            pltpu.async_copy(srcs.at[pl.ds(nb, 2 * GB)],
                             s_v.at[1 - pslot], esem)
            pltpu.async_copy(dsts.at[pl.ds(nb, 2 * GB)],
                             d_v.at[1 - pslot], esem)
            pltpu.async_copy(typs.at[pl.ds(nb, 2 * GB)],
                             t_v.at[1 - pslot], esem)

        gdesc = []
        for s in range(2):
            for j in range(GB // L):
                el = pl.ds(s * GB + j * L, L)
                sl = pl.ds(j * L, L)
                tv = t_v[pslot, el]
                t = jnp.minimum(tv, R - 1)
                gidx[s, sl] = t * N_PAD + s_v[pslot, el]
                widx[s, sl] = jnp.minimum(tv * N_PAD + d_v[pslot, el], RN - 1)
                didx[s, sl] = d_v[pslot, el]
            rslot = rows_v.at[pl.ds(s * GB, GB)]
            gdesc.append((
                pltpu.async_copy(tbl.at[gidx.at[s]], rslot, gsem),
                pltpu.async_copy(rc.at[widx.at[s]],
                                 w_v.at[pl.ds(s * GB, GB)], ssem),
            ))

        for s in range(2):
            dg, dw = gdesc[s]
            dg.wait()
            dw.wait()

            def gbody(g, carry, s=s):
                wvec = w_v[pl.ds(s * GB + g * L, L)]
                for jj in range(L):
                    j = s * GB + g * L + jj
                    wv = wvec[jj] * ones
                    for ccp in range(H // L):
                        sl = pl.ds(ccp * L, L)
                        rows_v[j, sl] = rows_v[j, sl] * wv
                return carry

            lax.fori_loop(0, GB // L, gbody, 0)
            pltpu.async_copy(rows_v.at[pl.ds(s * GB, GB)],
                             acc.at[didx.at[s]], wsem, add=True)
        return carry

    lax.fori_loop(0, npairs, bbody, 0)

    for s in range(2):
        pltpu.make_async_copy(rows_v.at[pl.ds(s * GB, GB)],
                              acc.at[didx.at[s]], wsem).wait()
    plsc.subcore_barrier()

    pltpu.sync_copy(acc.at[pl.ds(sid * RPT, RPT)],
                    agg_out.at[cid, pl.ds(sid * RPT, RPT)])


_sc_agg = pl.kernel(
    _sc_agg_body,
    out_type=jax.ShapeDtypeStruct((NC, N_PAD, H), jnp.float32),
    mesh=_MESH,
    scratch_types=[
        pltpu.VMEM((2, 2 * GB), jnp.int32),
        pltpu.VMEM((2, 2 * GB), jnp.int32),
        pltpu.VMEM((2, 2 * GB), jnp.int32),
        pltpu.VMEM((2, GB), jnp.int32),
        pltpu.VMEM((2, GB), jnp.int32),
        pltpu.VMEM((2, GB), jnp.int32),
        pltpu.VMEM((2 * GB,), jnp.float32),
        pltpu.VMEM((2 * GB, H), jnp.float32),
        pltpu.VMEM_SHARED((ACC_ROWS, H), jnp.float32),
        pltpu.SemaphoreType.DMA,
        pltpu.SemaphoreType.DMA,
        pltpu.SemaphoreType.DMA,
        pltpu.SemaphoreType.DMA,
    ],
)


def _tc_recip(cnt):
    rpb = 64

    def body(c_ref, o_ref):
        c = c_ref[0] + c_ref[1]
        o_ref[...] = 1.0 / jnp.maximum(c, 1.0)

    return pl.pallas_call(
        body,
        grid=(RN // 128 // rpb,),
        in_specs=[pl.BlockSpec((NC, rpb, 128), lambda i: (0, i, 0))],
        out_specs=pl.BlockSpec((rpb, 128), lambda i: (i, 0)),
        out_shape=jax.ShapeDtypeStruct((RN // 128, 128), jnp.float32),
    )(cnt)


def _tc_ht(h_pad, W):
    def body(h_ref, w_ref, o_ref):
        for r in range(R):
            o_ref[r] = jnp.dot(h_ref[...], w_ref[r],
                               preferred_element_type=jnp.float32)

    return pl.pallas_call(
        body,
        grid=(NBLK,),
        in_specs=[
            pl.BlockSpec((BM, H), lambda i: (i, 0)),
            pl.BlockSpec((R, H, H), lambda i: (0, 0, 0)),
        ],
        out_specs=pl.BlockSpec((R, BM, H), lambda i: (0, i, 0)),
        out_shape=jax.ShapeDtypeStruct((R, N_PAD, H), jnp.float32),
    )(h_pad, W)


def _tc_layer(h_pad, agg, root, b):
    def body(h_ref, a_ref, root_ref, b_ref, o_ref):
        acc = jnp.dot(h_ref[...], root_ref[...],
                      preferred_element_type=jnp.float32) + b_ref[...]
        acc = acc + a_ref[0] + a_ref[1]
        o_ref[...] = jnp.maximum(acc, 0.0)

    return pl.pallas_call(
        body,
        grid=(NBLK,),
        in_specs=[
            pl.BlockSpec((BM, H), lambda i: (i, 0)),
            pl.BlockSpec((NC, BM, H), lambda i: (0, i, 0)),
            pl.BlockSpec((H, H), lambda i: (0, 0)),
            pl.BlockSpec((1, H), lambda i: (0, 0)),
        ],
        out_specs=pl.BlockSpec((BM, H), lambda i: (i, 0)),
        out_shape=jax.ShapeDtypeStruct((N_PAD, H), jnp.float32),
    )(h_pad, agg, root, b.reshape(1, H))


def _tc_final(h1, agg, root2, b2, batch3, lin1_w, lin1_b, lin2_w, lin2_b):
    def body(h_ref, a_ref, root_ref, b_ref, bt_ref,
             l1w_ref, l1b_ref, l2w_ref, l2b_ref, o_ref, pool, pcnt):
        i = pl.program_id(0)
        acc = jnp.dot(h_ref[...], root_ref[...],
                      preferred_element_type=jnp.float32) + b_ref[...]
        h2 = jnp.maximum(acc + a_ref[0] + a_ref[1], 0.0)

        bt = bt_ref[0, 0, :]
        onehot = (bt[:, None] ==
                  lax.broadcasted_iota(jnp.int32, (BM, G), 1)
                  ).astype(jnp.float32)

        @pl.when(i == 0)
        def _():
            pool[...] = jnp.zeros((G, H), jnp.float32)
            pcnt[...] = jnp.zeros((G, H), jnp.float32)

        dn = (((0,), (0,)), ((), ()))
        pool[...] += lax.dot_general(onehot, h2, dn,
                                     preferred_element_type=jnp.float32)
        pcnt[...] += lax.dot_general(onehot, jnp.ones((BM, H), jnp.float32),
                                     dn, preferred_element_type=jnp.float32)

        @pl.when(i == NBLK - 1)
        def _():
            pooled = pool[...] / jnp.maximum(pcnt[...], 1.0)
            hh = jnp.maximum(
                jnp.dot(pooled, l1w_ref[...],
                        preferred_element_type=jnp.float32) + l1b_ref[...],
                0.0)
            o_ref[...] = jnp.dot(hh, l2w_ref[...],
                                 preferred_element_type=jnp.float32) + l2b_ref[...]

    return pl.pallas_call(
        body,
        grid=(NBLK,),
        in_specs=[
            pl.BlockSpec((BM, H), lambda i: (i, 0)),
            pl.BlockSpec((NC, BM, H), lambda i: (0, i, 0)),
            pl.BlockSpec((H, H), lambda i: (0, 0)),
            pl.BlockSpec((1, H), lambda i: (0, 0)),
            pl.BlockSpec((1, 1, BM), lambda i: (i, 0, 0)),
            pl.BlockSpec((H, H), lambda i: (0, 0)),
            pl.BlockSpec((1, H), lambda i: (0, 0)),
            pl.BlockSpec((H, C), lambda i: (0, 0)),
            pl.BlockSpec((1, C), lambda i: (0, 0)),
        ],
        out_specs=pl.BlockSpec((G, C), lambda i: (0, 0)),
        out_shape=jax.ShapeDtypeStruct((G, C), jnp.float32),
        scratch_shapes=[
            pltpu.VMEM((G, H), jnp.float32),
            pltpu.VMEM((G, H), jnp.float32),
        ],
    )(h1, agg, root2, b2.reshape(1, H), batch3,
      lin1_w, lin1_b.reshape(1, H), lin2_w, lin2_b.reshape(1, C))


def kernel(x, edge_index, edge_type, batch, W1, root1, b1, W2, root2, b2,
           lin1_w, lin1_b, lin2_w, lin2_b):
    x_pad = jnp.pad(x, ((0, N_PAD - N), (0, 0)))
    epad = E_PAD - E
    srcs = jnp.pad(edge_index[0], (0, epad))
    dsts = jnp.pad(edge_index[1], (0, epad))
    typs = jnp.pad(edge_type, (0, epad), constant_values=R)
    dsts_agg = jnp.pad(edge_index[1], (0, epad), constant_values=N_PAD)
    batch3 = jnp.pad(batch, (0, N_PAD - N),
                     constant_values=G).reshape(NBLK, 1, BM)
    zs = jnp.zeros((RPT + 16, H), jnp.float32)
    zc = jnp.zeros((CPT + 128,), jnp.float32)

    cnt = _sc_cnt(dsts, typs, zc)
    rc = _tc_recip(cnt.reshape(NC, RN // 128, 128)).reshape(RN)

    ht1 = _tc_ht(x_pad, W1).reshape(RN, H)
    agg1 = _sc_agg(ht1, rc, srcs, dsts_agg, typs, zs)
    h1 = _tc_layer(x_pad, agg1, root1, b1)

    ht2 = _tc_ht(h1, W2).reshape(RN, H)
    agg2 = _sc_agg(ht2, rc, srcs, dsts_agg, typs, zs)
    return _tc_final(h1, agg2, root2, b2, batch3,
                     lin1_w, lin1_b, lin2_w, lin2_b)
